# baseline (speedup 1.0000x reference)
import functools

import jax
import jax.numpy as jnp
from jax import lax
from jax.experimental import pallas as pl
from jax.experimental.pallas import tpu as pltpu
from jax.experimental.pallas import tpu_sc as plsc

N_NODES = 10000
N_EDGES = 320000
DIN = 128
DH = 16
H = 8
D1 = H * DH
DOUT = 64
DF = 64

NC = 2
NS = 16
NW = NC * NS
NP = 10240
ROWS_PER_TILE = NP // NS
EB = 80
EDGES_PER_W = N_EDGES // NW
NBATCH = EDGES_PER_W // EB

_MESH = plsc.VectorSubcoreMesh(core_axis_name="c", subcore_axis_name="s",
                               num_cores=NC, num_subcores=NS)


_GATHER_DNUMS = lax.GatherDimensionNumbers(
    offset_dims=(), collapsed_slice_dims=(0,), start_index_map=(0,))


def _lane_rotate(v, idx):
    return lax.gather(v, idx[:, None], _GATHER_DNUMS, (1,),
                      mode=lax.GatherScatterMode.PROMISE_IN_BOUNDS)


def _all_reduce_lanes(v):
    for sh in (8, 4, 2, 1):
        idx = (lax.iota(jnp.int32, 16) + sh) & 15
        v = v + _lane_rotate(v, idx)
    return v


def _tree4_pack(t0, t1, t2, t3, lane):
    r8 = (lane + 8) & 15
    i4 = (lane & 8) | ((lane + 4) & 7)
    i2 = (lane & 12) | ((lane + 2) & 3)
    i1 = (lane & 14) | ((lane + 1) & 1)
    mlt8 = lane < 8
    mq = (lane & 4) == 0
    u0 = t0 + _lane_rotate(t0, r8)
    u1 = t1 + _lane_rotate(t1, r8)
    u2 = t2 + _lane_rotate(t2, r8)
    u3 = t3 + _lane_rotate(t3, r8)
    m01 = jnp.where(mlt8, u0, _lane_rotate(u1, r8))
    m23 = jnp.where(mlt8, u2, _lane_rotate(u3, r8))
    m01 = m01 + _lane_rotate(m01, i4)
    m23 = m23 + _lane_rotate(m23, i4)
    mm = jnp.where(mq, m01, _lane_rotate(m23, i4))
    mm = mm + _lane_rotate(mm, i2)
    return mm + _lane_rotate(mm, i1)


def _edge_kernel_body(nheads, xl_hbm, xr_hbm, src_hbm, dst_hbm,
                      att_hbm, num_out, den_out,
                      rows_l, rows_r, wbuf, sidx, didx, att_v, num_s, den_s,
                      gsl, gsr):
    nblk = DF // 16
    cid = lax.axis_index("c")
    sid = lax.axis_index("s")
    wid = sid * NC + cid

    def zrow(i, _):
        for q in range(nblk):
            rows_l[i, pl.ds(16 * q, 16)] = jnp.zeros((16,), jnp.float32)
        wbuf[i, :] = jnp.zeros((16,), jnp.float32)
        return 0
    lax.fori_loop(0, EB, zrow, 0)

    def zcopy(k, _):
        off = sid * ROWS_PER_TILE + k * EB
        pltpu.sync_copy(rows_l, num_s.at[pl.ds(off, EB)])
        pltpu.sync_copy(wbuf, den_s.at[pl.ds(off, EB)])
        return 0
    lax.fori_loop(0, ROWS_PER_TILE // EB, zcopy, 0)
    plsc.subcore_barrier()

    pltpu.sync_copy(att_hbm, att_v)
    att_vecs = [att_v[q, :] for q in range(nblk)]
    lane = lax.iota(jnp.int32, 16)
    bph = nblk // nheads

    pltpu.sync_copy(src_hbm.at[wid], sidx)
    pltpu.sync_copy(dst_hbm.at[wid], didx)

    def batch_body(b, _):
        cp1 = pltpu.async_copy(xl_hbm.at[sidx.at[b]], rows_l, gsl)
        cp2 = pltpu.async_copy(xr_hbm.at[didx.at[b]], rows_r, gsr)
        cp1.wait()
        cp2.wait()

        if nheads == 4:
            remap = ((lane & 1) * 8) | ((lane & 2) * 2)
            splats = [jnp.full((16,), p, jnp.int32) for p in (0, 8, 4, 12)]

            def edge_body(e, _):
                ts, lvs = [], []
                for h in range(4):
                    lv = rows_l[e, pl.ds(16 * h, 16)]
                    rv = rows_r[e, pl.ds(16 * h, 16)]
                    lvs.append(lv)
                    s = lv + rv
                    ls = jnp.maximum(s, 0.2 * s)
                    ts.append(ls * att_vecs[h])
                w = jnp.exp(_tree4_pack(ts[0], ts[1], ts[2], ts[3], lane))
                wbuf[e, :] = _lane_rotate(w, remap)
                for h in range(4):
                    wh = _lane_rotate(w, splats[h])
                    rows_l[e, pl.ds(16 * h, 16)] = lvs[h] * wh
                return 0
        else:
            def edge_body(e, _):
                hacc = jnp.zeros((16,), jnp.float32)
                lvs = []
                for q in range(bph):
                    lv = rows_l[e, pl.ds(16 * q, 16)]
                    rv = rows_r[e, pl.ds(16 * q, 16)]
                    lvs.append(lv)
                    s = lv + rv
                    ls = jnp.maximum(s, 0.2 * s)
                    hacc = hacc + ls * att_vecs[q]
                wsplat = jnp.exp(_all_reduce_lanes(hacc))
                for q in range(bph):
                    rows_l[e, pl.ds(16 * q, 16)] = lvs[q] * wsplat
                wbuf[e, :] = wsplat
                return 0

        lax.fori_loop(0, EB, edge_body, 0, unroll=2)
        pltpu.sync_copy(rows_l, num_s.at[didx.at[b]], add=True)
        pltpu.sync_copy(wbuf, den_s.at[didx.at[b]], add=True)
        return 0

    lax.fori_loop(0, NBATCH, batch_body, 0)
    plsc.subcore_barrier()

    def ocopy(k, _):
        off = sid * ROWS_PER_TILE + k * EB
        pltpu.sync_copy(num_s.at[pl.ds(off, EB)],
                        num_out.at[cid].at[pl.ds(off, EB)])
        pltpu.sync_copy(den_s.at[pl.ds(off, EB)],
                        den_out.at[cid].at[pl.ds(off, EB)])
        return 0
    lax.fori_loop(0, ROWS_PER_TILE // EB, ocopy, 0)


def _make_edge_kernel(nheads):
    body = functools.partial(_edge_kernel_body, nheads)
    return pl.kernel(
        body,
        out_type=(
            jax.ShapeDtypeStruct((NC, NP, DF), jnp.float32),
            jax.ShapeDtypeStruct((NC, NP, 16), jnp.float32),
        ),
        mesh=_MESH,
        compiler_params=pltpu.CompilerParams(use_tc_tiling_on_sc=False),
        scratch_types=[
            pltpu.VMEM((EB, DF), jnp.float32),
            pltpu.VMEM((EB, DF), jnp.float32),
            pltpu.VMEM((EB, 16), jnp.float32),
            pltpu.VMEM((NBATCH, EB), jnp.int32),
            pltpu.VMEM((NBATCH, EB), jnp.int32),
            pltpu.VMEM((DF // 16, 16), jnp.float32),
            pltpu.VMEM_SHARED((NP, DF), jnp.float32),
            pltpu.VMEM_SHARED((NP, 16), jnp.float32),
            pltpu.SemaphoreType.DMA,
            pltpu.SemaphoreType.DMA,
        ],
    )


_edge_l1 = _make_edge_kernel(4)
_edge_l2 = _make_edge_kernel(1)


def _proj1_body(x_ref, wl_ref, bl_ref, wr_ref, br_ref,
                xla_ref, xlb_ref, xra_ref, xrb_ref):
    xb = x_ref[...]
    dn = (((1,), (1,)), ((), ()))
    xl = lax.dot_general(xb, wl_ref[...], dn,
                         preferred_element_type=jnp.float32) + bl_ref[...]
    xr = lax.dot_general(xb, wr_ref[...], dn,
                         preferred_element_type=jnp.float32) + br_ref[...]
    xla_ref[...] = xl[:, :DF]
    xlb_ref[...] = xl[:, DF:]
    xra_ref[...] = xr[:, :DF]
    xrb_ref[...] = xr[:, DF:]


def _fuse_body(na0_ref, na1_ref, nb0_ref, nb1_ref,
               da0_ref, da1_ref, db0_ref, db1_ref, bias_ref,
               wl_ref, bl_ref, wr_ref, br_ref, xl_ref, xr_ref):
    numa = na0_ref[0] + na1_ref[0]
    numb = nb0_ref[0] + nb1_ref[0]
    dena = da0_ref[0] + da1_ref[0]
    denb = db0_ref[0] + db1_ref[0]
    num = jnp.concatenate([numa, numb], axis=1)
    den8 = jnp.concatenate([dena[:, :4], denb[:, :4]], axis=1)
    colh = lax.broadcasted_iota(jnp.int32, (8, D1), 1) // DH
    rowh = lax.broadcasted_iota(jnp.int32, (8, D1), 0)
    expand = (colh == rowh).astype(jnp.float32)
    denr = lax.dot_general(den8, expand, (((1,), (0,)), ((), ())),
                           preferred_element_type=jnp.float32)
    h = num / (denr + 1e-16) + bias_ref[...]
    h = jnp.where(h > 0, h, jnp.exp(h) - 1.0)
    dn = (((1,), (1,)), ((), ()))
    xl_ref[...] = lax.dot_general(h, wl_ref[...], dn,
                                  preferred_element_type=jnp.float32) + bl_ref[...]
    xr_ref[...] = lax.dot_general(h, wr_ref[...], dn,
                                  preferred_element_type=jnp.float32) + br_ref[...]


def _final_body(numa_ref, numb_ref, dena_ref, denb_ref, bias_ref, out_ref):
    num = numa_ref[0] + numb_ref[0]
    den = dena_ref[0] + denb_ref[0]
    out_ref[...] = num / (den[:, 0:1] + 1e-16) + bias_ref[...]


def kernel(x, edge_index, Wl1, bl1, Wr1, br1, att1, bias1,
           Wl2, bl2, Wr2, br2, att2, bias2):
    src = edge_index[0].reshape(NW, NBATCH, EB)
    dst = edge_index[1].reshape(NW, NBATCH, EB)

    grid1 = N_NODES // 1000
    xla, xlb, xra, xrb = pl.pallas_call(
        _proj1_body,
        grid=(grid1,),
        in_specs=[
            pl.BlockSpec((1000, DIN), lambda i: (i, 0)),
            pl.BlockSpec((D1, DIN), lambda i: (0, 0)),
            pl.BlockSpec((1, D1), lambda i: (0, 0)),
            pl.BlockSpec((D1, DIN), lambda i: (0, 0)),
            pl.BlockSpec((1, D1), lambda i: (0, 0)),
        ],
        out_specs=[pl.BlockSpec((1000, DF), lambda i: (i, 0))] * 4,
        out_shape=[jax.ShapeDtypeStruct((N_NODES, DF), jnp.float32)] * 4,
    )(x, Wl1, bl1.reshape(1, D1), Wr1, br1.reshape(1, D1))

    numa, dena = _edge_l1(xla, xra, src, dst, att1[:4])
    numb, denb = _edge_l1(xlb, xrb, src, dst, att1[4:])

    grid2 = NP // 1024
    xl2, xr2 = pl.pallas_call(
        _fuse_body,
        grid=(grid2,),
        in_specs=[
            pl.BlockSpec((1, 1024, DF), lambda i: (0, i, 0)),
            pl.BlockSpec((1, 1024, DF), lambda i: (1, i, 0)),
            pl.BlockSpec((1, 1024, DF), lambda i: (0, i, 0)),
            pl.BlockSpec((1, 1024, DF), lambda i: (1, i, 0)),
            pl.BlockSpec((1, 1024, 16), lambda i: (0, i, 0)),
            pl.BlockSpec((1, 1024, 16), lambda i: (1, i, 0)),
            pl.BlockSpec((1, 1024, 16), lambda i: (0, i, 0)),
            pl.BlockSpec((1, 1024, 16), lambda i: (1, i, 0)),
            pl.BlockSpec((1, D1), lambda i: (0, 0)),
            pl.BlockSpec((DOUT, D1), lambda i: (0, 0)),
            pl.BlockSpec((1, DOUT), lambda i: (0, 0)),
            pl.BlockSpec((DOUT, D1), lambda i: (0, 0)),
            pl.BlockSpec((1, DOUT), lambda i: (0, 0)),
        ],
        out_specs=[
            pl.BlockSpec((1024, DOUT), lambda i: (i, 0)),
            pl.BlockSpec((1024, DOUT), lambda i: (i, 0)),
        ],
        out_shape=[
            jax.ShapeDtypeStruct((NP, DOUT), jnp.float32),
            jax.ShapeDtypeStruct((NP, DOUT), jnp.float32),
        ],
    )(numa, numa, numb, numb, dena, dena, denb, denb, bias1.reshape(1, D1),
      Wl2, bl2.reshape(1, DOUT), Wr2, br2.reshape(1, DOUT))

    num2, den2 = _edge_l2(xl2, xr2, src, dst, att2.reshape(DOUT // 16, 16))

    out = pl.pallas_call(
        _final_body,
        grid=(grid2,),
        in_specs=[
            pl.BlockSpec((1, 1024, DOUT), lambda i: (0, i, 0)),
            pl.BlockSpec((1, 1024, DOUT), lambda i: (1, i, 0)),
            pl.BlockSpec((1, 1024, 16), lambda i: (0, i, 0)),
            pl.BlockSpec((1, 1024, 16), lambda i: (1, i, 0)),
            pl.BlockSpec((1, DOUT), lambda i: (0, 0)),
        ],
        out_specs=pl.BlockSpec((1024, DOUT), lambda i: (i, 0)),
        out_shape=jax.ShapeDtypeStruct((NP, DOUT), jnp.float32),
    )(num2, num2, den2, den2, bias2.reshape(1, DOUT))

    return out[:N_NODES]


# speedup vs baseline: 1.0144x; 1.0144x over previous
import functools

import jax
import jax.numpy as jnp
from jax import lax
from jax.experimental import pallas as pl
from jax.experimental.pallas import tpu as pltpu
from jax.experimental.pallas import tpu_sc as plsc

N_NODES = 10000
N_EDGES = 320000
DIN = 128
DH = 16
H = 8
D1 = H * DH
DOUT = 64
DF = 64

NC = 2
NS = 16
NW = NC * NS
NP = 10240
ROWS_PER_TILE = NP // NS
EB = 80
EDGES_PER_W = N_EDGES // NW
NBATCH = EDGES_PER_W // EB

_MESH = plsc.VectorSubcoreMesh(core_axis_name="c", subcore_axis_name="s",
                               num_cores=NC, num_subcores=NS)


_GATHER_DNUMS = lax.GatherDimensionNumbers(
    offset_dims=(), collapsed_slice_dims=(0,), start_index_map=(0,))


def _lane_rotate(v, idx):
    return lax.gather(v, idx[:, None], _GATHER_DNUMS, (1,),
                      mode=lax.GatherScatterMode.PROMISE_IN_BOUNDS)


def _all_reduce_lanes(v):
    for sh in (8, 4, 2, 1):
        idx = (lax.iota(jnp.int32, 16) + sh) & 15
        v = v + _lane_rotate(v, idx)
    return v


def _tree4_pack(t0, t1, t2, t3, lane):
    r8 = (lane + 8) & 15
    i4 = (lane & 8) | ((lane + 4) & 7)
    i2 = (lane & 12) | ((lane + 2) & 3)
    i1 = (lane & 14) | ((lane + 1) & 1)
    mlt8 = lane < 8
    mq = (lane & 4) == 0
    u0 = t0 + _lane_rotate(t0, r8)
    u1 = t1 + _lane_rotate(t1, r8)
    u2 = t2 + _lane_rotate(t2, r8)
    u3 = t3 + _lane_rotate(t3, r8)
    m01 = jnp.where(mlt8, u0, _lane_rotate(u1, r8))
    m23 = jnp.where(mlt8, u2, _lane_rotate(u3, r8))
    m01 = m01 + _lane_rotate(m01, i4)
    m23 = m23 + _lane_rotate(m23, i4)
    mm = jnp.where(mq, m01, _lane_rotate(m23, i4))
    mm = mm + _lane_rotate(mm, i2)
    return mm + _lane_rotate(mm, i1)


def _edge_kernel_body(nheads, xl_hbm, xr_hbm, src_hbm, dst_hbm,
                      att_hbm, num_out, den_out,
                      rows_l, rows_r, wbuf, sidx, didx, att_v, num_s, den_s,
                      gsl, gsr):
    nblk = DF // 16
    cid = lax.axis_index("c")
    sid = lax.axis_index("s")
    wid = sid * NC + cid

    def zrow(i, _):
        for q in range(nblk):
            rows_l[i, pl.ds(16 * q, 16)] = jnp.zeros((16,), jnp.float32)
        wbuf[i, :] = jnp.zeros((16,), jnp.float32)
        return 0
    lax.fori_loop(0, EB, zrow, 0)

    def zcopy(k, _):
        off = sid * ROWS_PER_TILE + k * EB
        pltpu.sync_copy(rows_l, num_s.at[pl.ds(off, EB)])
        pltpu.sync_copy(wbuf, den_s.at[pl.ds(off, EB)])
        return 0
    lax.fori_loop(0, ROWS_PER_TILE // EB, zcopy, 0)
    plsc.subcore_barrier()

    pltpu.sync_copy(att_hbm, att_v)
    att_vecs = [att_v[q, :] for q in range(nblk)]
    lane = lax.iota(jnp.int32, 16)
    bph = nblk // nheads

    pltpu.sync_copy(src_hbm.at[wid], sidx)
    pltpu.sync_copy(dst_hbm.at[wid], didx)

    def batch_body(b, _):
        cp1 = pltpu.async_copy(xl_hbm.at[sidx.at[b]], rows_l, gsl)
        cp2 = pltpu.async_copy(xr_hbm.at[didx.at[b]], rows_r, gsr)
        cp1.wait()
        cp2.wait()

        if nheads == 4:
            remap = ((lane & 1) * 8) | ((lane & 2) * 2)
            splats = [jnp.full((16,), p, jnp.int32) for p in (0, 8, 4, 12)]

            def edge_body(e, _):
                ts, lvs = [], []
                for h in range(4):
                    lv = rows_l[e, pl.ds(16 * h, 16)]
                    rv = rows_r[e, pl.ds(16 * h, 16)]
                    lvs.append(lv)
                    s = lv + rv
                    ls = jnp.maximum(s, 0.2 * s)
                    ts.append(ls * att_vecs[h])
                w = jnp.exp(_tree4_pack(ts[0], ts[1], ts[2], ts[3], lane))
                wbuf[e, :] = _lane_rotate(w, remap)
                for h in range(4):
                    wh = _lane_rotate(w, splats[h])
                    rows_l[e, pl.ds(16 * h, 16)] = lvs[h] * wh
                return 0
        else:
            def edge_body(e, _):
                hacc = jnp.zeros((16,), jnp.float32)
                lvs = []
                for q in range(bph):
                    lv = rows_l[e, pl.ds(16 * q, 16)]
                    rv = rows_r[e, pl.ds(16 * q, 16)]
                    lvs.append(lv)
                    s = lv + rv
                    ls = jnp.maximum(s, 0.2 * s)
                    hacc = hacc + ls * att_vecs[q]
                wsplat = jnp.exp(_all_reduce_lanes(hacc))
                for q in range(bph):
                    rows_l[e, pl.ds(16 * q, 16)] = lvs[q] * wsplat
                wbuf[e, :] = wsplat
                return 0

        lax.fori_loop(0, EB, edge_body, 0)
        pltpu.sync_copy(rows_l, num_s.at[didx.at[b]], add=True)
        pltpu.sync_copy(wbuf, den_s.at[didx.at[b]], add=True)
        return 0

    lax.fori_loop(0, NBATCH, batch_body, 0)
    plsc.subcore_barrier()

    def ocopy(k, _):
        off = sid * ROWS_PER_TILE + k * EB
        pltpu.sync_copy(num_s.at[pl.ds(off, EB)],
                        num_out.at[cid].at[pl.ds(off, EB)])
        pltpu.sync_copy(den_s.at[pl.ds(off, EB)],
                        den_out.at[cid].at[pl.ds(off, EB)])
        return 0
    lax.fori_loop(0, ROWS_PER_TILE // EB, ocopy, 0)


def _make_edge_kernel(nheads):
    body = functools.partial(_edge_kernel_body, nheads)
    return pl.kernel(
        body,
        out_type=(
            jax.ShapeDtypeStruct((NC, NP, DF), jnp.float32),
            jax.ShapeDtypeStruct((NC, NP, 16), jnp.float32),
        ),
        mesh=_MESH,
        compiler_params=pltpu.CompilerParams(use_tc_tiling_on_sc=False),
        scratch_types=[
            pltpu.VMEM((EB, DF), jnp.float32),
            pltpu.VMEM((EB, DF), jnp.float32),
            pltpu.VMEM((EB, 16), jnp.float32),
            pltpu.VMEM((NBATCH, EB), jnp.int32),
            pltpu.VMEM((NBATCH, EB), jnp.int32),
            pltpu.VMEM((DF // 16, 16), jnp.float32),
            pltpu.VMEM_SHARED((NP, DF), jnp.float32),
            pltpu.VMEM_SHARED((NP, 16), jnp.float32),
            pltpu.SemaphoreType.DMA,
            pltpu.SemaphoreType.DMA,
        ],
    )


_edge_l1 = _make_edge_kernel(4)
_edge_l2 = _make_edge_kernel(1)


def _proj1_body(x_ref, wl_ref, bl_ref, wr_ref, br_ref,
                xla_ref, xlb_ref, xra_ref, xrb_ref):
    xb = x_ref[...]
    dn = (((1,), (1,)), ((), ()))
    xl = lax.dot_general(xb, wl_ref[...], dn,
                         preferred_element_type=jnp.float32) + bl_ref[...]
    xr = lax.dot_general(xb, wr_ref[...], dn,
                         preferred_element_type=jnp.float32) + br_ref[...]
    xla_ref[...] = xl[:, :DF]
    xlb_ref[...] = xl[:, DF:]
    xra_ref[...] = xr[:, :DF]
    xrb_ref[...] = xr[:, DF:]


def _fuse_body(na0_ref, na1_ref, nb0_ref, nb1_ref,
               da0_ref, da1_ref, db0_ref, db1_ref, bias_ref,
               wl_ref, bl_ref, wr_ref, br_ref, xl_ref, xr_ref):
    numa = na0_ref[0] + na1_ref[0]
    numb = nb0_ref[0] + nb1_ref[0]
    dena = da0_ref[0] + da1_ref[0]
    denb = db0_ref[0] + db1_ref[0]
    num = jnp.concatenate([numa, numb], axis=1)
    den8 = jnp.concatenate([dena[:, :4], denb[:, :4]], axis=1)
    colh = lax.broadcasted_iota(jnp.int32, (8, D1), 1) // DH
    rowh = lax.broadcasted_iota(jnp.int32, (8, D1), 0)
    expand = (colh == rowh).astype(jnp.float32)
    denr = lax.dot_general(den8, expand, (((1,), (0,)), ((), ())),
                           preferred_element_type=jnp.float32)
    h = num / (denr + 1e-16) + bias_ref[...]
    h = jnp.where(h > 0, h, jnp.exp(h) - 1.0)
    dn = (((1,), (1,)), ((), ()))
    xl_ref[...] = lax.dot_general(h, wl_ref[...], dn,
                                  preferred_element_type=jnp.float32) + bl_ref[...]
    xr_ref[...] = lax.dot_general(h, wr_ref[...], dn,
                                  preferred_element_type=jnp.float32) + br_ref[...]


def _final_body(numa_ref, numb_ref, dena_ref, denb_ref, bias_ref, out_ref):
    num = numa_ref[0] + numb_ref[0]
    den = dena_ref[0] + denb_ref[0]
    out_ref[...] = num / (den[:, 0:1] + 1e-16) + bias_ref[...]


def kernel(x, edge_index, Wl1, bl1, Wr1, br1, att1, bias1,
           Wl2, bl2, Wr2, br2, att2, bias2):
    src = edge_index[0].reshape(NW, NBATCH, EB)
    dst = edge_index[1].reshape(NW, NBATCH, EB)

    grid1 = N_NODES // 1000
    xla, xlb, xra, xrb = pl.pallas_call(
        _proj1_body,
        grid=(grid1,),
        in_specs=[
            pl.BlockSpec((1000, DIN), lambda i: (i, 0)),
            pl.BlockSpec((D1, DIN), lambda i: (0, 0)),
            pl.BlockSpec((1, D1), lambda i: (0, 0)),
            pl.BlockSpec((D1, DIN), lambda i: (0, 0)),
            pl.BlockSpec((1, D1), lambda i: (0, 0)),
        ],
        out_specs=[pl.BlockSpec((1000, DF), lambda i: (i, 0))] * 4,
        out_shape=[jax.ShapeDtypeStruct((N_NODES, DF), jnp.float32)] * 4,
    )(x, Wl1, bl1.reshape(1, D1), Wr1, br1.reshape(1, D1))

    numa, dena = _edge_l1(xla, xra, src, dst, att1[:4])
    numb, denb = _edge_l1(xlb, xrb, src, dst, att1[4:])

    grid2 = NP // 1024
    xl2, xr2 = pl.pallas_call(
        _fuse_body,
        grid=(grid2,),
        in_specs=[
            pl.BlockSpec((1, 1024, DF), lambda i: (0, i, 0)),
            pl.BlockSpec((1, 1024, DF), lambda i: (1, i, 0)),
            pl.BlockSpec((1, 1024, DF), lambda i: (0, i, 0)),
            pl.BlockSpec((1, 1024, DF), lambda i: (1, i, 0)),
            pl.BlockSpec((1, 1024, 16), lambda i: (0, i, 0)),
            pl.BlockSpec((1, 1024, 16), lambda i: (1, i, 0)),
            pl.BlockSpec((1, 1024, 16), lambda i: (0, i, 0)),
            pl.BlockSpec((1, 1024, 16), lambda i: (1, i, 0)),
            pl.BlockSpec((1, D1), lambda i: (0, 0)),
            pl.BlockSpec((DOUT, D1), lambda i: (0, 0)),
            pl.BlockSpec((1, DOUT), lambda i: (0, 0)),
            pl.BlockSpec((DOUT, D1), lambda i: (0, 0)),
            pl.BlockSpec((1, DOUT), lambda i: (0, 0)),
        ],
        out_specs=[
            pl.BlockSpec((1024, DOUT), lambda i: (i, 0)),
            pl.BlockSpec((1024, DOUT), lambda i: (i, 0)),
        ],
        out_shape=[
            jax.ShapeDtypeStruct((NP, DOUT), jnp.float32),
            jax.ShapeDtypeStruct((NP, DOUT), jnp.float32),
        ],
    )(numa, numa, numb, numb, dena, dena, denb, denb, bias1.reshape(1, D1),
      Wl2, bl2.reshape(1, DOUT), Wr2, br2.reshape(1, DOUT))

    num2, den2 = _edge_l2(xl2, xr2, src, dst, att2.reshape(DOUT // 16, 16))

    out = pl.pallas_call(
        _final_body,
        grid=(grid2,),
        in_specs=[
            pl.BlockSpec((1, 1024, DOUT), lambda i: (0, i, 0)),
            pl.BlockSpec((1, 1024, DOUT), lambda i: (1, i, 0)),
            pl.BlockSpec((1, 1024, 16), lambda i: (0, i, 0)),
            pl.BlockSpec((1, 1024, 16), lambda i: (1, i, 0)),
            pl.BlockSpec((1, DOUT), lambda i: (0, 0)),
        ],
        out_specs=pl.BlockSpec((1024, DOUT), lambda i: (i, 0)),
        out_shape=jax.ShapeDtypeStruct((NP, DOUT), jnp.float32),
    )(num2, num2, den2, den2, bias2.reshape(1, DOUT))

    return out[:N_NODES]
